# Initial kernel scaffold; baseline (speedup 1.0000x reference)
import jax
import jax.numpy as jnp
from jax.experimental import pallas as pl


def kernel(x, segment_ids, W):
    raise NotImplementedError("write your pallas kernel here")


# speedup vs baseline: 5.4451x; 5.4451x over previous
import functools

import jax
import jax.numpy as jnp
from jax.experimental import pallas as pl
from jax.experimental.pallas import tpu as pltpu

_NSEG = 16


def _pool_kernel(seg_ref, x_ref, wt_ref, out_ref, m_ref, d_ref, *, nb):
    i = pl.program_id(0)
    nseg = m_ref.shape[1]

    @pl.when(i == 0)
    def _init():
        m_ref[...] = jnp.full(m_ref.shape, -jnp.inf, jnp.float32)
        d_ref[...] = jnp.zeros(d_ref.shape, jnp.float32)
        out_ref[...] = jnp.zeros(out_ref.shape, jnp.float32)

    x = x_ref[...]
    ids = seg_ref[0]
    scores = jax.lax.dot_general(
        x, wt_ref[...], (((1,), (0,)), ((), ())),
        preferred_element_type=jnp.float32)

    rows = x.shape[0]
    lane = jax.lax.broadcasted_iota(jnp.int32, (rows, nseg), 1)
    onehot = lane == ids

    neg_inf = jnp.float32(-jnp.inf)
    bm = jnp.max(jnp.where(onehot, scores, neg_inf), axis=0, keepdims=True)
    m_old = m_ref[...]
    m_new = jnp.maximum(m_old, bm)
    alpha = jnp.exp(jnp.where(m_new == neg_inf, 0.0, m_old - m_new))

    mrow = jnp.sum(jnp.where(onehot, m_new, 0.0), axis=1, keepdims=True)
    e = jnp.exp(scores - mrow)
    we = jnp.where(onehot, e, 0.0)

    d_ref[...] = alpha * d_ref[...] + jnp.sum(we, axis=0, keepdims=True)
    m_ref[...] = m_new

    contrib = jax.lax.dot_general(
        we, x, (((0,), (0,)), ((), ())),
        preferred_element_type=jnp.float32)
    alpha_col = alpha.reshape(nseg, 1)
    out_ref[...] = out_ref[...] * alpha_col + contrib

    @pl.when(i == nb - 1)
    def _finish():
        d = d_ref[...]
        denom = jnp.where(d > 0.0, d, 1.0).reshape(nseg, 1)
        out_ref[...] = out_ref[...] / denom


@jax.jit
def _attn_pool(x, segment_ids, W):
    n, d = x.shape
    rows = 2048
    nb = n // rows
    ids = segment_ids.astype(jnp.int32).reshape(nb, rows, 1)
    wt = W.reshape(d, 1)
    return pl.pallas_call(
        functools.partial(_pool_kernel, nb=nb),
        grid=(nb,),
        in_specs=[
            pl.BlockSpec((1, rows, 1), lambda i: (i, 0, 0)),
            pl.BlockSpec((rows, d), lambda i: (i, 0)),
            pl.BlockSpec((d, 1), lambda i: (0, 0)),
        ],
        out_specs=pl.BlockSpec((_NSEG, d), lambda i: (0, 0)),
        out_shape=jax.ShapeDtypeStruct((_NSEG, d), jnp.float32),
        scratch_shapes=[
            pltpu.VMEM((1, _NSEG), jnp.float32),
            pltpu.VMEM((1, _NSEG), jnp.float32),
        ],
        compiler_params=pltpu.CompilerParams(
            dimension_semantics=("arbitrary",)),
    )(ids, x, wt)


def kernel(x, segment_ids, W):
    return _attn_pool(x, segment_ids, W)
